# baseline (speedup 1.0000x reference)
import functools

import jax
import jax.numpy as jnp
from jax import lax
from jax.experimental import pallas as pl
from jax.experimental.pallas import tpu as pltpu
from jax.experimental.pallas import tpu_sc as plsc

L_PAD = 56
SENT_BLOCK = 64
GATHER_CHUNK = 128
NBUF = 4
LOOKAHEAD = 2
PACK = 64
PAIR = 2 * L_PAD


def _sc_gather(table_u32, inds):
    rows_total = inds.shape[0]
    info = plsc.get_sparse_core_info()
    num_workers = info.num_cores * info.num_subcores
    per_worker = rows_total // num_workers
    n_chunks = per_worker // GATHER_CHUNK
    mesh = plsc.VectorSubcoreMesh(core_axis_name="c", subcore_axis_name="s")

    @functools.partial(
        pl.kernel,
        mesh=mesh,
        out_type=jax.ShapeDtypeStruct((rows_total, PACK), jnp.uint32),
        scratch_types=[
            pltpu.VMEM((per_worker,), jnp.int32),
            [pltpu.VMEM((GATHER_CHUNK, PACK), jnp.uint32)] * NBUF,
            [pltpu.SemaphoreType.DMA] * NBUF,
            [pltpu.SemaphoreType.DMA] * NBUF,
            pltpu.SemaphoreType.DMA,
        ],
        compiler_params=pltpu.CompilerParams(use_tc_tiling_on_sc=False),
    )
    def gather_kernel(table_hbm, idx_hbm, out_hbm, idx_v, bufs, gsems, wsems,
                      isem):
        wid = lax.axis_index("s") * info.num_cores + lax.axis_index("c")
        base = wid * per_worker
        pltpu.async_copy(idx_hbm.at[pl.ds(base, per_worker)], idx_v,
                         isem).wait()

        def start_gather(chunk, b):
            pltpu.async_copy(
                table_hbm.at[idx_v.at[pl.ds(chunk * GATHER_CHUNK,
                                            GATHER_CHUNK)]],
                bufs[b], gsems[b])

        def wait_gather(b):
            pltpu.make_async_copy(
                table_hbm.at[idx_v.at[pl.ds(0, GATHER_CHUNK)]],
                bufs[b], gsems[b]).wait()

        def start_write(chunk, b):
            pltpu.async_copy(
                bufs[b],
                out_hbm.at[pl.ds(base + chunk * GATHER_CHUNK, GATHER_CHUNK)],
                wsems[b])

        def wait_write(b):
            pltpu.make_async_copy(
                bufs[b], out_hbm.at[pl.ds(0, GATHER_CHUNK)],
                wsems[b]).wait()

        for j in range(LOOKAHEAD):
            start_gather(j, j % NBUF)

        def group(g, carry):
            for u in range(NBUF):
                j = g * NBUF + u
                b = u
                bg = (u + LOOKAHEAD) % NBUF

                @pl.when(j < n_chunks)
                def _():
                    @pl.when(j + LOOKAHEAD < n_chunks)
                    def _():
                        @pl.when(j + LOOKAHEAD >= NBUF)
                        def _():
                            wait_write(bg)
                        start_gather(j + LOOKAHEAD, bg)

                    wait_gather(b)
                    start_write(j, b)
            return carry

        n_groups = (n_chunks + NBUF - 1) // NBUF
        lax.fori_loop(0, n_groups, group, 0)
        for b in range(min(NBUF, n_chunks)):
            wait_write(b)

    return gather_kernel(table_u32, inds)


def _tc_body(x_ref, e_ref, W_ref, b_ref, g_ref, be_ref, Wa_ref, ba_ref,
             ctx_ref, out_ref, aw_ref):
    B = SENT_BLOCK
    W = W_ref[...]
    b = b_ref[...]
    g = g_ref[...]
    be = be_ref[...]
    Wa = Wa_ref[...]
    ba = ba_ref[...]
    ctxv = ctx_ref[...]

    xq = x_ref[...]
    e0 = lax.bitcast_convert_type(xq << 16, jnp.float32)
    e1 = lax.bitcast_convert_type(xq & jnp.uint32(0xFFFF0000), jnp.float32)
    x_even = jnp.concatenate([e0[:, :64], e1[:, :64]], axis=1)
    x_odd = jnp.concatenate([e0[:, 64:], e1[:, 64:]], axis=1)
    h_even = lax.dot_general(x_even.astype(jnp.bfloat16), W,
                             (((1,), (0,)), ((), ())),
                             preferred_element_type=jnp.float32)
    h_odd = lax.dot_general(x_odd.astype(jnp.bfloat16), W,
                            (((1,), (0,)), ((), ())),
                            preferred_element_type=jnp.float32)
    h_even = h_even.astype(jnp.bfloat16)
    h_odd = h_odd.astype(jnp.bfloat16)

    e = e_ref[...]
    src_all = e[:, 0, :]
    dst_all = e[:, 1, :]

    iota_r = lax.broadcasted_iota(jnp.int32, (PAIR, 2 * 128), 0)
    iota_e = lax.broadcasted_iota(jnp.int32, (PAIR, 2 * 128), 1)
    eoff = jnp.where(iota_e >= 128, L_PAD, 0)
    qvec = jnp.where(iota_r < L_PAD, 2 * iota_r, 2 * iota_r - (PAIR - 1))
    io_i = lax.broadcasted_iota(jnp.int32, (PAIR, PAIR), 0)
    io_c = lax.broadcasted_iota(jnp.int32, (PAIR, PAIR), 1)
    qcol = jnp.where(io_c < L_PAD, 2 * io_c, 2 * io_c - (PAIR - 1))
    eyeq = (io_i == qcol).astype(jnp.float32)
    eyeq_bf = eyeq.astype(jnp.bfloat16)

    words = []
    for p in range(B // 2):
        s0, s1 = 2 * p, 2 * p + 1
        src_pair = jnp.concatenate(
            [src_all[s0:s0 + 1, :], src_all[s1:s1 + 1, :]], axis=1) + eoff[:1]
        dst_pair = jnp.concatenate(
            [dst_all[s0:s0 + 1, :], dst_all[s1:s1 + 1, :]], axis=1) + eoff[:1]
        a_dst = (iota_r == dst_pair).astype(jnp.float32)
        a_srcq = (qvec == src_pair).astype(jnp.float32)
        deg = 1.0 + jnp.sum(a_dst, axis=1, keepdims=True)
        dinv = lax.rsqrt(deg)
        dinv_q = lax.dot_general(eyeq_bf, dinv.astype(jnp.bfloat16),
                                 (((0,), (0,)), ((), ())),
                                 preferred_element_type=jnp.float32)
        a_dst_bf = a_dst.astype(jnp.bfloat16)
        a_srcq_n = (a_srcq * dinv_q).astype(jnp.bfloat16)
        m = lax.dot_general(a_dst_bf, a_srcq_n, (((1,), (1,)), ((), ())),
                            preferred_element_type=jnp.float32)
        m = ((m + eyeq * dinv) * dinv).astype(jnp.bfloat16)
        h_pair = jnp.concatenate(
            [h_even[L_PAD * p:L_PAD * (p + 1), :],
             h_odd[L_PAD * p:L_PAD * (p + 1), :]], axis=0)
        words.append(lax.dot_general(m, h_pair, (((1,), (0,)), ((), ())),
                                     preferred_element_type=jnp.float32))
    word2 = jnp.concatenate(words, axis=0) + b

    mu = jnp.mean(word2, axis=1, keepdims=True)
    cen = word2 - mu
    var = jnp.mean(cen * cen, axis=1, keepdims=True)
    normed = (cen * lax.rsqrt(var + 1e-5) * g + be).astype(jnp.bfloat16)

    ah = jnp.tanh(lax.dot_general(normed, Wa, (((1,), (0,)), ((), ())),
                                  preferred_element_type=jnp.float32) + ba)
    sc2 = lax.dot_general(ah.astype(jnp.bfloat16), ctxv,
                          (((1,), (0,)), ((), ())),
                          preferred_element_type=jnp.float32)
    sc3 = sc2.reshape(B, L_PAD)
    neg = jnp.float32(-1e30)
    lane = lax.broadcasted_iota(jnp.int32, (B, L_PAD), 1)
    scm = jnp.where(lane < 50, sc3, neg)
    mx = jnp.max(scm, axis=1, keepdims=True)
    ex = jnp.exp(scm - mx)
    aw = ex / jnp.sum(ex, axis=1, keepdims=True)

    word3 = word2.reshape(B, L_PAD, 128)
    aw50 = aw[:, :50]
    out_ref[...] = word3[:, :50, :] * aw50[:, :, None]
    aw_ref[...] = aw50


def _tc_compute(xq2, edges, W, b, g, be, Wa, ba, ctxv):
    S = edges.shape[0]
    grid = (S // SENT_BLOCK,)
    rows_per_block = SENT_BLOCK * L_PAD // 2
    return pl.pallas_call(
        _tc_body,
        grid=grid,
        in_specs=[
            pl.BlockSpec((rows_per_block, 128), lambda i: (i, 0)),
            pl.BlockSpec((SENT_BLOCK, 2, 128), lambda i: (i, 0, 0)),
            pl.BlockSpec((128, 128), lambda i: (0, 0)),
            pl.BlockSpec((1, 128), lambda i: (0, 0)),
            pl.BlockSpec((1, 128), lambda i: (0, 0)),
            pl.BlockSpec((1, 128), lambda i: (0, 0)),
            pl.BlockSpec((128, 64), lambda i: (0, 0)),
            pl.BlockSpec((1, 64), lambda i: (0, 0)),
            pl.BlockSpec((64, 1), lambda i: (0, 0)),
        ],
        out_specs=(
            pl.BlockSpec((SENT_BLOCK, 50, 128), lambda i: (i, 0, 0)),
            pl.BlockSpec((SENT_BLOCK, 50), lambda i: (i, 0)),
        ),
        out_shape=(
            jax.ShapeDtypeStruct((S, 50, 128), jnp.float32),
            jax.ShapeDtypeStruct((S, 50), jnp.float32),
        ),
        compiler_params=pltpu.CompilerParams(
            dimension_semantics=("arbitrary",),
        ),
    )(xq2, edges, W, b, g, be, Wa, ba, ctxv)


def kernel(sents, code_lenth, word_edge, table, W_gcn, b_gcn, ln_gamma,
           ln_beta, W_att, b_att, ctx):
    S, L = sents.shape
    V, E = table.shape
    inds = jnp.pad(sents, ((0, 0), (0, L_PAD - L)))
    table_u32 = lax.bitcast_convert_type(
        table.astype(jnp.bfloat16).reshape(V, PACK, 2), jnp.uint32)
    perm = jnp.concatenate([jnp.arange(0, E, 2), jnp.arange(1, E, 2)])
    W_perm = W_gcn[perm, :].astype(jnp.bfloat16)
    n_chunks = 2
    sk = S // n_chunks
    outs, aws = [], []
    for k in range(n_chunks):
        inds_k = inds[k * sk:(k + 1) * sk].reshape(-1)
        xu = _sc_gather(table_u32, inds_k)
        xq2 = xu.reshape(sk * L_PAD // 2, 128)
        o, a = _tc_compute(
            xq2, word_edge[k * sk:(k + 1) * sk], W_perm,
            b_gcn.reshape(1, -1), ln_gamma.reshape(1, -1),
            ln_beta.reshape(1, -1),
            W_att.astype(jnp.bfloat16), b_att.reshape(1, -1),
            ctx.reshape(-1, 1).astype(jnp.bfloat16))
        outs.append(o)
        aws.append(a)
    return (jnp.concatenate(outs, axis=0), jnp.concatenate(aws, axis=0))


# speedup vs baseline: 1.1422x; 1.1422x over previous
import functools

import jax
import jax.numpy as jnp
from jax import lax
from jax.experimental import pallas as pl
from jax.experimental.pallas import tpu as pltpu
from jax.experimental.pallas import tpu_sc as plsc

L_PAD = 56
SENT_BLOCK = 128
GATHER_CHUNK = 128
NBUF = 4
LOOKAHEAD = 2
PACK = 64
PAIR = 2 * L_PAD


def _sc_gather(table_u32, inds):
    rows_total = inds.shape[0]
    info = plsc.get_sparse_core_info()
    num_workers = info.num_cores * info.num_subcores
    per_worker = rows_total // num_workers
    n_chunks = per_worker // GATHER_CHUNK
    mesh = plsc.VectorSubcoreMesh(core_axis_name="c", subcore_axis_name="s")

    @functools.partial(
        pl.kernel,
        mesh=mesh,
        out_type=jax.ShapeDtypeStruct((rows_total, PACK), jnp.uint32),
        scratch_types=[
            pltpu.VMEM((per_worker,), jnp.int32),
            [pltpu.VMEM((GATHER_CHUNK, PACK), jnp.uint32)] * NBUF,
            [pltpu.SemaphoreType.DMA] * NBUF,
            [pltpu.SemaphoreType.DMA] * NBUF,
            pltpu.SemaphoreType.DMA,
        ],
        compiler_params=pltpu.CompilerParams(use_tc_tiling_on_sc=False),
    )
    def gather_kernel(table_hbm, idx_hbm, out_hbm, idx_v, bufs, gsems, wsems,
                      isem):
        wid = lax.axis_index("s") * info.num_cores + lax.axis_index("c")
        base = wid * per_worker
        pltpu.async_copy(idx_hbm.at[pl.ds(base, per_worker)], idx_v,
                         isem).wait()

        def start_gather(chunk, b):
            pltpu.async_copy(
                table_hbm.at[idx_v.at[pl.ds(chunk * GATHER_CHUNK,
                                            GATHER_CHUNK)]],
                bufs[b], gsems[b])

        def wait_gather(b):
            pltpu.make_async_copy(
                table_hbm.at[idx_v.at[pl.ds(0, GATHER_CHUNK)]],
                bufs[b], gsems[b]).wait()

        def start_write(chunk, b):
            pltpu.async_copy(
                bufs[b],
                out_hbm.at[pl.ds(base + chunk * GATHER_CHUNK, GATHER_CHUNK)],
                wsems[b])

        def wait_write(b):
            pltpu.make_async_copy(
                bufs[b], out_hbm.at[pl.ds(0, GATHER_CHUNK)],
                wsems[b]).wait()

        for j in range(LOOKAHEAD):
            start_gather(j, j % NBUF)

        def group(g, carry):
            for u in range(NBUF):
                j = g * NBUF + u
                b = u
                bg = (u + LOOKAHEAD) % NBUF

                @pl.when(j < n_chunks)
                def _():
                    @pl.when(j + LOOKAHEAD < n_chunks)
                    def _():
                        @pl.when(j + LOOKAHEAD >= NBUF)
                        def _():
                            wait_write(bg)
                        start_gather(j + LOOKAHEAD, bg)

                    wait_gather(b)
                    start_write(j, b)
            return carry

        n_groups = (n_chunks + NBUF - 1) // NBUF
        lax.fori_loop(0, n_groups, group, 0)
        for b in range(min(NBUF, n_chunks)):
            wait_write(b)

    return gather_kernel(table_u32, inds)


def _tc_body(x_ref, e_ref, W_ref, b_ref, g_ref, be_ref, Wa_ref, ba_ref,
             ctx_ref, out_ref, aw_ref):
    B = SENT_BLOCK
    W = W_ref[...]
    b = b_ref[...]
    g = g_ref[...]
    be = be_ref[...]
    Wa = Wa_ref[...]
    ba = ba_ref[...]
    ctxv = ctx_ref[...]

    xq = x_ref[...]
    e0 = lax.bitcast_convert_type(xq << 16, jnp.float32)
    e1 = lax.bitcast_convert_type(xq & jnp.uint32(0xFFFF0000), jnp.float32)
    x_even = jnp.concatenate([e0[:, :64], e1[:, :64]], axis=1)
    x_odd = jnp.concatenate([e0[:, 64:], e1[:, 64:]], axis=1)
    h_even = lax.dot_general(x_even.astype(jnp.bfloat16), W,
                             (((1,), (0,)), ((), ())),
                             preferred_element_type=jnp.float32)
    h_odd = lax.dot_general(x_odd.astype(jnp.bfloat16), W,
                            (((1,), (0,)), ((), ())),
                            preferred_element_type=jnp.float32)
    h_even = h_even.astype(jnp.bfloat16)
    h_odd = h_odd.astype(jnp.bfloat16)

    e = e_ref[...]
    src_all = e[:, 0, :]
    dst_all = e[:, 1, :]

    iota_r = lax.broadcasted_iota(jnp.int32, (PAIR, 2 * 128), 0)
    iota_e = lax.broadcasted_iota(jnp.int32, (PAIR, 2 * 128), 1)
    eoff = jnp.where(iota_e >= 128, L_PAD, 0)
    qvec = jnp.where(iota_r < L_PAD, 2 * iota_r, 2 * iota_r - (PAIR - 1))
    io_i = lax.broadcasted_iota(jnp.int32, (PAIR, PAIR), 0)
    io_c = lax.broadcasted_iota(jnp.int32, (PAIR, PAIR), 1)
    qcol = jnp.where(io_c < L_PAD, 2 * io_c, 2 * io_c - (PAIR - 1))
    eyeq = (io_i == qcol).astype(jnp.float32)

    words = []
    for p in range(B // 2):
        s0, s1 = 2 * p, 2 * p + 1
        src_pair = jnp.concatenate(
            [src_all[s0:s0 + 1, :], src_all[s1:s1 + 1, :]], axis=1) + eoff[:1]
        dst_pair = jnp.concatenate(
            [dst_all[s0:s0 + 1, :], dst_all[s1:s1 + 1, :]], axis=1) + eoff[:1]
        a_dst = (iota_r == dst_pair).astype(jnp.float32)
        a_srcq = (qvec == src_pair).astype(jnp.float32)
        deg = 1.0 + jnp.sum(a_dst, axis=1, keepdims=True)
        deg_q = 1.0 + jnp.sum((qvec == dst_pair).astype(jnp.float32), axis=1,
                              keepdims=True)
        dinv = lax.rsqrt(deg)
        dinv_q = lax.rsqrt(deg_q)
        a_dst_n = (a_dst * dinv).astype(jnp.bfloat16)
        a_srcq_n = (a_srcq * dinv_q).astype(jnp.bfloat16)
        m = lax.dot_general(a_dst_n, a_srcq_n, (((1,), (1,)), ((), ())),
                            preferred_element_type=jnp.float32)
        m = (m + eyeq * (dinv * dinv)).astype(jnp.bfloat16)
        h_pair = jnp.concatenate(
            [h_even[L_PAD * p:L_PAD * (p + 1), :],
             h_odd[L_PAD * p:L_PAD * (p + 1), :]], axis=0)
        words.append(lax.dot_general(m, h_pair, (((1,), (0,)), ((), ())),
                                     preferred_element_type=jnp.float32))
    word2 = jnp.concatenate(words, axis=0) + b

    mu = jnp.mean(word2, axis=1, keepdims=True)
    cen = word2 - mu
    var = jnp.mean(cen * cen, axis=1, keepdims=True)
    normed = (cen * lax.rsqrt(var + 1e-5) * g + be).astype(jnp.bfloat16)

    ah = jnp.tanh(lax.dot_general(normed, Wa, (((1,), (0,)), ((), ())),
                                  preferred_element_type=jnp.float32) + ba)
    sc2 = lax.dot_general(ah.astype(jnp.bfloat16), ctxv,
                          (((1,), (0,)), ((), ())),
                          preferred_element_type=jnp.float32)
    sc3 = sc2.reshape(B, L_PAD)
    neg = jnp.float32(-1e30)
    lane = lax.broadcasted_iota(jnp.int32, (B, L_PAD), 1)
    scm = jnp.where(lane < 50, sc3, neg)
    mx = jnp.max(scm, axis=1, keepdims=True)
    ex = jnp.exp(scm - mx)
    aw = ex / jnp.sum(ex, axis=1, keepdims=True)

    word3 = word2.reshape(B, L_PAD, 128)
    aw50 = aw[:, :50]
    out_ref[...] = word3[:, :50, :] * aw50[:, :, None]
    aw_ref[...] = aw50


def _tc_compute(xq2, edges, W, b, g, be, Wa, ba, ctxv):
    S = edges.shape[0]
    grid = (S // SENT_BLOCK,)
    rows_per_block = SENT_BLOCK * L_PAD // 2
    return pl.pallas_call(
        _tc_body,
        grid=grid,
        in_specs=[
            pl.BlockSpec((rows_per_block, 128), lambda i: (i, 0)),
            pl.BlockSpec((SENT_BLOCK, 2, 128), lambda i: (i, 0, 0)),
            pl.BlockSpec((128, 128), lambda i: (0, 0)),
            pl.BlockSpec((1, 128), lambda i: (0, 0)),
            pl.BlockSpec((1, 128), lambda i: (0, 0)),
            pl.BlockSpec((1, 128), lambda i: (0, 0)),
            pl.BlockSpec((128, 64), lambda i: (0, 0)),
            pl.BlockSpec((1, 64), lambda i: (0, 0)),
            pl.BlockSpec((64, 1), lambda i: (0, 0)),
        ],
        out_specs=(
            pl.BlockSpec((SENT_BLOCK, 50, 128), lambda i: (i, 0, 0)),
            pl.BlockSpec((SENT_BLOCK, 50), lambda i: (i, 0)),
        ),
        out_shape=(
            jax.ShapeDtypeStruct((S, 50, 128), jnp.float32),
            jax.ShapeDtypeStruct((S, 50), jnp.float32),
        ),
        compiler_params=pltpu.CompilerParams(
            dimension_semantics=("arbitrary",),
        ),
    )(xq2, edges, W, b, g, be, Wa, ba, ctxv)


def kernel(sents, code_lenth, word_edge, table, W_gcn, b_gcn, ln_gamma,
           ln_beta, W_att, b_att, ctx):
    S, L = sents.shape
    V, E = table.shape
    inds = jnp.pad(sents, ((0, 0), (0, L_PAD - L)))
    table_u32 = lax.bitcast_convert_type(
        table.astype(jnp.bfloat16).reshape(V, PACK, 2), jnp.uint32)
    perm = jnp.concatenate([jnp.arange(0, E, 2), jnp.arange(1, E, 2)])
    W_perm = W_gcn[perm, :].astype(jnp.bfloat16)
    n_chunks = 2
    sk = S // n_chunks
    outs, aws = [], []
    for k in range(n_chunks):
        inds_k = inds[k * sk:(k + 1) * sk].reshape(-1)
        xu = _sc_gather(table_u32, inds_k)
        xq2 = xu.reshape(sk * L_PAD // 2, 128)
        o, a = _tc_compute(
            xq2, word_edge[k * sk:(k + 1) * sk], W_perm,
            b_gcn.reshape(1, -1), ln_gamma.reshape(1, -1),
            ln_beta.reshape(1, -1),
            W_att.astype(jnp.bfloat16), b_att.reshape(1, -1),
            ctx.reshape(-1, 1).astype(jnp.bfloat16))
        outs.append(o)
        aws.append(a)
    return (jnp.concatenate(outs, axis=0), jnp.concatenate(aws, axis=0))
